# baseline (speedup 1.0000x reference)
import functools

import jax
import jax.numpy as jnp
from jax.experimental import pallas as pl
from jax.experimental.pallas import tpu as pltpu

_H1 = 0.5
_H2 = 0.5


def _blk(n, target):
    for b in range(min(target, n), 7, -1):
        if n % b == 0 and b % 8 == 0:
            return b
    return n


def _feat_kernel(s1_ref, s2_ref, w1_ref, w2_ref, b1_ref, b2_ref, f_ref):
    s1 = s1_ref[...]
    s2 = s2_ref[...]
    w1 = w1_ref[...]
    w2 = w2_ref[...]
    f11 = jnp.dot(s1, w1, preferred_element_type=jnp.float32) + b1_ref[...]
    f21 = jnp.dot(s2, w1, preferred_element_type=jnp.float32) + b1_ref[...]
    f12 = jnp.dot(s1, w2, preferred_element_type=jnp.float32) + b2_ref[...]
    f22 = jnp.dot(s2, w2, preferred_element_type=jnp.float32) + b2_ref[...]
    f_ref[...] = jnp.concatenate([f11, f21, f12, f22], axis=1).astype(jnp.bfloat16)


def _main_kernel(nm, bm, adj_ref, f_ref, av1_ref, av2_ref, mskt_ref, wd_ref,
                 sb1_ref, sb2_ref, o1_ref, o2_ref,
                 h1_sc, h2_sc, craw_sc):
    t = pl.program_id(0)
    nh = av1_ref.shape[1]

    @pl.when(t < nm)
    def _():
        a = adj_ref[...].astype(jnp.bfloat16)
        g = jax.lax.dot_general(
            a, f_ref[...], (((1,), (0,)), ((), ())),
            preferred_element_type=jnp.float32)
        a1 = av1_ref[...]
        a2 = av2_ref[...]
        g11 = g[:, 0 * nh:1 * nh]
        g21 = g[:, 1 * nh:2 * nh]
        g12 = g[:, 2 * nh:3 * nh]
        g22 = g[:, 3 * nh:4 * nh]
        p11 = jnp.where(g11 > 0, g11, a1 * g11)
        p21 = jnp.where(g21 > 0, g21, a1 * g21)
        p12 = jnp.where(g12 > 0, g12, a2 * g12)
        p22 = jnp.where(g22 > 0, g22, a2 * g22)
        h1v = p11 + _H2 * p22
        h2v = p21 + _H1 * p12
        row = t * bm
        h1_sc[pl.ds(row, bm), :] = h1v.astype(jnp.bfloat16)
        h2_sc[pl.ds(row, bm), :] = h2v.astype(jnp.bfloat16)
        part = jnp.sum(h1v * mskt_ref[pl.ds(row, bm), :], axis=0, keepdims=True)

        @pl.when(t == 0)
        def _():
            craw_sc[...] = jnp.zeros_like(craw_sc)

        craw_sc[...] += part

    @pl.when(t == nm)
    def _():
        c = jax.nn.sigmoid(craw_sc[...] / jnp.sum(mskt_ref[...]))
        v = jax.lax.dot_general(
            c, wd_ref[...], (((1,), (1,)), ((), ())),
            preferred_element_type=jnp.float32)
        h1 = h1_sc[...].astype(jnp.float32)
        h2 = h2_sc[...].astype(jnp.float32)
        o1_ref[...] = jnp.sum(h1 * v, axis=1, keepdims=True) + sb1_ref[...]
        o2_ref[...] = jnp.sum(h2 * v, axis=1, keepdims=True) + sb2_ref[...]


def kernel(seq1, seq2, adj, sparse, training, msk, samp_bias1, samp_bias2,
           W1, b1, a1, W2, b2, a2, Wd, bd):
    n = seq1.shape[1]
    d = seq1.shape[2]
    nh = W1.shape[1]
    s1 = seq1[0]
    s2 = seq2[0]
    A = adj[0]

    bm_f = _blk(n, 2500)
    F = pl.pallas_call(
        _feat_kernel,
        grid=(n // bm_f,),
        in_specs=[
            pl.BlockSpec((bm_f, d), lambda i: (i, 0)),
            pl.BlockSpec((bm_f, d), lambda i: (i, 0)),
            pl.BlockSpec((d, nh), lambda i: (0, 0)),
            pl.BlockSpec((d, nh), lambda i: (0, 0)),
            pl.BlockSpec((1, nh), lambda i: (0, 0)),
            pl.BlockSpec((1, nh), lambda i: (0, 0)),
        ],
        out_specs=pl.BlockSpec((bm_f, 4 * nh), lambda i: (i, 0)),
        out_shape=jax.ShapeDtypeStruct((n, 4 * nh), jnp.bfloat16),
    )(s1, s2, W1, W2, b1.reshape(1, nh), b2.reshape(1, nh))

    bm = _blk(n, 200)
    nm = n // bm
    av1 = jnp.full((1, nh), a1, jnp.float32)
    av2 = jnp.full((1, nh), a2, jnp.float32)
    mskt = msk.reshape(n, 1)
    sb1 = (samp_bias1 + bd[0]).reshape(n, 1)
    sb2 = (samp_bias2 + bd[0]).reshape(n, 1)
    o1, o2 = pl.pallas_call(
        functools.partial(_main_kernel, nm, bm),
        grid=(nm + 1,),
        in_specs=[
            pl.BlockSpec((bm, n), lambda t: (jnp.minimum(t, nm - 1), 0)),
            pl.BlockSpec((n, 4 * nh), lambda t: (0, 0)),
            pl.BlockSpec((1, nh), lambda t: (0, 0)),
            pl.BlockSpec((1, nh), lambda t: (0, 0)),
            pl.BlockSpec((n, 1), lambda t: (0, 0)),
            pl.BlockSpec((nh, nh), lambda t: (0, 0)),
            pl.BlockSpec((n, 1), lambda t: (0, 0)),
            pl.BlockSpec((n, 1), lambda t: (0, 0)),
        ],
        out_specs=[
            pl.BlockSpec((n, 1), lambda t: (0, 0)),
            pl.BlockSpec((n, 1), lambda t: (0, 0)),
        ],
        out_shape=[
            jax.ShapeDtypeStruct((n, 1), jnp.float32),
            jax.ShapeDtypeStruct((n, 1), jnp.float32),
        ],
        scratch_shapes=[
            pltpu.VMEM((n, nh), jnp.bfloat16),
            pltpu.VMEM((n, nh), jnp.bfloat16),
            pltpu.VMEM((1, nh), jnp.float32),
        ],
        compiler_params=pltpu.CompilerParams(
            dimension_semantics=("arbitrary",)),
    )(A, F, av1, av2, mskt, Wd, sb1, sb2)

    return jnp.concatenate([o1.reshape(1, n), o2.reshape(1, n)], axis=1)


# speedup vs baseline: 1.2439x; 1.2439x over previous
import functools

import jax
import jax.numpy as jnp
from jax.experimental import pallas as pl
from jax.experimental.pallas import tpu as pltpu

_H1 = 0.5
_H2 = 0.5


def _blk(n, target):
    for b in range(min(target, n), 7, -1):
        if n % b == 0 and b % 8 == 0:
            return b
    return n


def _feat_kernel(s1_ref, s2_ref, w1_ref, w2_ref, b1_ref, b2_ref, f_ref):
    s1 = s1_ref[...]
    s2 = s2_ref[...]
    w1 = w1_ref[...]
    w2 = w2_ref[...]
    f11 = jnp.dot(s1, w1, preferred_element_type=jnp.float32) + b1_ref[...]
    f21 = jnp.dot(s2, w1, preferred_element_type=jnp.float32) + b1_ref[...]
    f12 = jnp.dot(s1, w2, preferred_element_type=jnp.float32) + b2_ref[...]
    f22 = jnp.dot(s2, w2, preferred_element_type=jnp.float32) + b2_ref[...]
    f_ref[...] = jnp.concatenate([f11, f21, f12, f22], axis=1).astype(jnp.bfloat16)


def _main_kernel(nm, bm, adj_ref, f_ref, av1_ref, av2_ref, mskb_ref, msk_ref,
                 wd_ref, o1_ref, o2_ref, h1_sc, h2_sc, craw_sc):
    t = pl.program_id(0)
    nh = av1_ref.shape[1]

    @pl.when(t < nm)
    def _():
        a = adj_ref[...].astype(jnp.bfloat16)
        g = jax.lax.dot_general(
            a, f_ref[...], (((1,), (0,)), ((), ())),
            preferred_element_type=jnp.float32)
        a1 = av1_ref[...]
        a2 = av2_ref[...]
        g11 = g[:, 0 * nh:1 * nh]
        g21 = g[:, 1 * nh:2 * nh]
        g12 = g[:, 2 * nh:3 * nh]
        g22 = g[:, 3 * nh:4 * nh]
        p11 = jnp.where(g11 > 0, g11, a1 * g11)
        p21 = jnp.where(g21 > 0, g21, a1 * g21)
        p12 = jnp.where(g12 > 0, g12, a2 * g12)
        p22 = jnp.where(g22 > 0, g22, a2 * g22)
        h1v = p11 + _H2 * p22
        h2v = p21 + _H1 * p12
        row = t * bm
        h1_sc[pl.ds(row, bm), :] = h1v.astype(jnp.bfloat16)
        h2_sc[pl.ds(row, bm), :] = h2v.astype(jnp.bfloat16)
        part = jax.lax.dot_general(
            mskb_ref[0], h1v, (((1,), (0,)), ((), ())),
            preferred_element_type=jnp.float32)

        @pl.when(t == 0)
        def _():
            craw_sc[...] = jnp.zeros_like(craw_sc)

        craw_sc[...] += part

    @pl.when(t == nm)
    def _():
        c = jax.nn.sigmoid(craw_sc[...] / jnp.sum(msk_ref[...]))
        v = jax.lax.dot_general(
            c, wd_ref[...], (((1,), (1,)), ((), ())),
            preferred_element_type=jnp.float32).astype(jnp.bfloat16)
        o1_ref[...] = jax.lax.dot_general(
            v, h1_sc[...], (((1,), (1,)), ((), ())),
            preferred_element_type=jnp.float32)
        o2_ref[...] = jax.lax.dot_general(
            v, h2_sc[...], (((1,), (1,)), ((), ())),
            preferred_element_type=jnp.float32)


def kernel(seq1, seq2, adj, sparse, training, msk, samp_bias1, samp_bias2,
           W1, b1, a1, W2, b2, a2, Wd, bd):
    n = seq1.shape[1]
    d = seq1.shape[2]
    nh = W1.shape[1]
    s1 = seq1[0]
    s2 = seq2[0]
    A = adj[0]

    bm_f = _blk(n, 2500)
    F = pl.pallas_call(
        _feat_kernel,
        grid=(n // bm_f,),
        in_specs=[
            pl.BlockSpec((bm_f, d), lambda i: (i, 0)),
            pl.BlockSpec((bm_f, d), lambda i: (i, 0)),
            pl.BlockSpec((d, nh), lambda i: (0, 0)),
            pl.BlockSpec((d, nh), lambda i: (0, 0)),
            pl.BlockSpec((1, nh), lambda i: (0, 0)),
            pl.BlockSpec((1, nh), lambda i: (0, 0)),
        ],
        out_specs=pl.BlockSpec((bm_f, 4 * nh), lambda i: (i, 0)),
        out_shape=jax.ShapeDtypeStruct((n, 4 * nh), jnp.bfloat16),
    )(s1, s2, W1, W2, b1.reshape(1, nh), b2.reshape(1, nh))

    bm = _blk(n, 400)
    nm = n // bm
    av1 = jnp.full((1, nh), a1, jnp.float32)
    av2 = jnp.full((1, nh), a2, jnp.float32)
    o1, o2 = pl.pallas_call(
        functools.partial(_main_kernel, nm, bm),
        grid=(nm + 1,),
        in_specs=[
            pl.BlockSpec((bm, n), lambda t: (jnp.minimum(t, nm - 1), 0)),
            pl.BlockSpec((n, 4 * nh), lambda t: (0, 0)),
            pl.BlockSpec((1, nh), lambda t: (0, 0)),
            pl.BlockSpec((1, nh), lambda t: (0, 0)),
            pl.BlockSpec((1, 1, bm), lambda t: (jnp.minimum(t, nm - 1), 0, 0)),
            pl.BlockSpec((1, n), lambda t: (0, 0)),
            pl.BlockSpec((nh, nh), lambda t: (0, 0)),
        ],
        out_specs=[
            pl.BlockSpec((1, n), lambda t: (0, 0)),
            pl.BlockSpec((1, n), lambda t: (0, 0)),
        ],
        out_shape=[
            jax.ShapeDtypeStruct((1, n), jnp.float32),
            jax.ShapeDtypeStruct((1, n), jnp.float32),
        ],
        scratch_shapes=[
            pltpu.VMEM((n, nh), jnp.bfloat16),
            pltpu.VMEM((n, nh), jnp.bfloat16),
            pltpu.VMEM((1, nh), jnp.float32),
        ],
        compiler_params=pltpu.CompilerParams(
            dimension_semantics=("arbitrary",)),
    )(A, F, av1, av2, msk.reshape(nm, 1, bm), msk, Wd)

    return jnp.concatenate(
        [o1 + samp_bias1 + bd[0], o2 + samp_bias2 + bd[0]], axis=1)


# speedup vs baseline: 1.3297x; 1.0690x over previous
import functools

import jax
import jax.numpy as jnp
from jax.experimental import pallas as pl
from jax.experimental.pallas import tpu as pltpu

_H1 = 0.5
_H2 = 0.5


def _blk(n, target):
    for b in range(min(target, n), 7, -1):
        if n % b == 0 and b % 8 == 0:
            return b
    return n


def _mega_kernel(nf, bmf, nm, bm,
                 s1_ref, s2_ref, w1_ref, w2_ref, b1_ref, b2_ref,
                 adj_ref, av1_ref, av2_ref, mskb_ref, msk_ref, wd_ref,
                 o1_ref, o2_ref,
                 f_sc, h1_sc, h2_sc, craw_sc):
    t = pl.program_id(0)
    nh = av1_ref.shape[1]

    @pl.when(t < nf)
    def _():
        s1 = s1_ref[...]
        s2 = s2_ref[...]
        w1 = w1_ref[...]
        w2 = w2_ref[...]
        f11 = jnp.dot(s1, w1, preferred_element_type=jnp.float32) + b1_ref[...]
        f21 = jnp.dot(s2, w1, preferred_element_type=jnp.float32) + b1_ref[...]
        f12 = jnp.dot(s1, w2, preferred_element_type=jnp.float32) + b2_ref[...]
        f22 = jnp.dot(s2, w2, preferred_element_type=jnp.float32) + b2_ref[...]
        f_sc[pl.ds(t * bmf, bmf), :] = jnp.concatenate(
            [f11, f21, f12, f22], axis=1).astype(jnp.bfloat16)

    @pl.when((t >= nf) & (t < nf + nm))
    def _():
        a = adj_ref[...].astype(jnp.bfloat16)
        g = jax.lax.dot_general(
            a, f_sc[...], (((1,), (0,)), ((), ())),
            preferred_element_type=jnp.float32)
        a1 = av1_ref[...]
        a2 = av2_ref[...]
        g11 = g[:, 0 * nh:1 * nh]
        g21 = g[:, 1 * nh:2 * nh]
        g12 = g[:, 2 * nh:3 * nh]
        g22 = g[:, 3 * nh:4 * nh]
        p11 = jnp.where(g11 > 0, g11, a1 * g11)
        p21 = jnp.where(g21 > 0, g21, a1 * g21)
        p12 = jnp.where(g12 > 0, g12, a2 * g12)
        p22 = jnp.where(g22 > 0, g22, a2 * g22)
        h1v = p11 + _H2 * p22
        h2v = p21 + _H1 * p12
        row = (t - nf) * bm
        h1_sc[pl.ds(row, bm), :] = h1v.astype(jnp.bfloat16)
        h2_sc[pl.ds(row, bm), :] = h2v.astype(jnp.bfloat16)
        part = jax.lax.dot_general(
            mskb_ref[0], h1v, (((1,), (0,)), ((), ())),
            preferred_element_type=jnp.float32)

        @pl.when(t == nf)
        def _():
            craw_sc[...] = jnp.zeros_like(craw_sc)

        craw_sc[...] += part

    @pl.when(t == nf + nm)
    def _():
        c = jax.nn.sigmoid(craw_sc[...] / jnp.sum(msk_ref[...]))
        v = jax.lax.dot_general(
            c, wd_ref[...], (((1,), (1,)), ((), ())),
            preferred_element_type=jnp.float32).astype(jnp.bfloat16)
        o1_ref[...] = jax.lax.dot_general(
            v, h1_sc[...], (((1,), (1,)), ((), ())),
            preferred_element_type=jnp.float32)
        o2_ref[...] = jax.lax.dot_general(
            v, h2_sc[...], (((1,), (1,)), ((), ())),
            preferred_element_type=jnp.float32)


def kernel(seq1, seq2, adj, sparse, training, msk, samp_bias1, samp_bias2,
           W1, b1, a1, W2, b2, a2, Wd, bd):
    n = seq1.shape[1]
    d = seq1.shape[2]
    nh = W1.shape[1]
    s1 = seq1[0]
    s2 = seq2[0]
    A = adj[0]

    bmf = _blk(n, 2500)
    nf = n // bmf
    bm = _blk(n, 400)
    nm = n // bm
    av1 = jnp.full((1, nh), a1, jnp.float32)
    av2 = jnp.full((1, nh), a2, jnp.float32)

    def _fidx(t):
        return (jnp.minimum(t, nf - 1), 0)

    def _aidx(t):
        return (jnp.minimum(jnp.maximum(t - nf, 0), nm - 1), 0)

    def _midx(t):
        return (jnp.minimum(jnp.maximum(t - nf, 0), nm - 1), 0, 0)

    o1, o2 = pl.pallas_call(
        functools.partial(_mega_kernel, nf, bmf, nm, bm),
        grid=(nf + nm + 1,),
        in_specs=[
            pl.BlockSpec((bmf, d), _fidx),
            pl.BlockSpec((bmf, d), _fidx),
            pl.BlockSpec((d, nh), lambda t: (0, 0)),
            pl.BlockSpec((d, nh), lambda t: (0, 0)),
            pl.BlockSpec((1, nh), lambda t: (0, 0)),
            pl.BlockSpec((1, nh), lambda t: (0, 0)),
            pl.BlockSpec((bm, n), _aidx),
            pl.BlockSpec((1, nh), lambda t: (0, 0)),
            pl.BlockSpec((1, nh), lambda t: (0, 0)),
            pl.BlockSpec((1, 1, bm), _midx),
            pl.BlockSpec((1, n), lambda t: (0, 0)),
            pl.BlockSpec((nh, nh), lambda t: (0, 0)),
        ],
        out_specs=[
            pl.BlockSpec((1, n), lambda t: (0, 0)),
            pl.BlockSpec((1, n), lambda t: (0, 0)),
        ],
        out_shape=[
            jax.ShapeDtypeStruct((1, n), jnp.float32),
            jax.ShapeDtypeStruct((1, n), jnp.float32),
        ],
        scratch_shapes=[
            pltpu.VMEM((n, 4 * nh), jnp.bfloat16),
            pltpu.VMEM((n, nh), jnp.bfloat16),
            pltpu.VMEM((n, nh), jnp.bfloat16),
            pltpu.VMEM((1, nh), jnp.float32),
        ],
        compiler_params=pltpu.CompilerParams(
            dimension_semantics=("arbitrary",)),
    )(s1, s2, W1, W2, b1.reshape(1, nh), b2.reshape(1, nh),
      A, av1, av2, msk.reshape(nm, 1, bm), msk, Wd)

    return jnp.concatenate(
        [o1 + samp_bias1 + bd[0], o2 + samp_bias2 + bd[0]], axis=1)


# speedup vs baseline: 1.3350x; 1.0039x over previous
import functools

import jax
import jax.numpy as jnp
from jax.experimental import pallas as pl
from jax.experimental.pallas import tpu as pltpu

_H1 = 0.5
_H2 = 0.5


def _blk(n, target):
    for b in range(min(target, n), 7, -1):
        if n % b == 0 and b % 8 == 0:
            return b
    return n


def _mega_kernel(nf, bmf, nm, bm,
                 s1_ref, s2_ref, w1_ref, w2_ref, b1_ref, b2_ref,
                 adj_ref, av1_ref, av2_ref, mskb_ref, msk_ref, wd_ref,
                 o1_ref, o2_ref,
                 f_sc, h1_sc, h2_sc, craw_sc):
    t = pl.program_id(0)
    nh = av1_ref.shape[1]

    @pl.when(t < nf)
    def _():
        s1 = s1_ref[...]
        s2 = s2_ref[...]
        w1 = w1_ref[...]
        w2 = w2_ref[...]
        nhh = w1_ref.shape[1]
        row = t * bmf
        f_sc[pl.ds(row, bmf), 0 * nhh:1 * nhh] = (
            jnp.dot(s1, w1, preferred_element_type=jnp.float32)
            + b1_ref[...]).astype(jnp.bfloat16)
        f_sc[pl.ds(row, bmf), 1 * nhh:2 * nhh] = (
            jnp.dot(s2, w1, preferred_element_type=jnp.float32)
            + b1_ref[...]).astype(jnp.bfloat16)
        f_sc[pl.ds(row, bmf), 2 * nhh:3 * nhh] = (
            jnp.dot(s1, w2, preferred_element_type=jnp.float32)
            + b2_ref[...]).astype(jnp.bfloat16)
        f_sc[pl.ds(row, bmf), 3 * nhh:4 * nhh] = (
            jnp.dot(s2, w2, preferred_element_type=jnp.float32)
            + b2_ref[...]).astype(jnp.bfloat16)

    @pl.when((t >= nf) & (t < nf + nm))
    def _():
        a = adj_ref[...].astype(jnp.bfloat16)
        g = jax.lax.dot_general(
            a, f_sc[...], (((1,), (0,)), ((), ())),
            preferred_element_type=jnp.float32)
        a1 = av1_ref[...]
        a2 = av2_ref[...]
        g11 = g[:, 0 * nh:1 * nh]
        g21 = g[:, 1 * nh:2 * nh]
        g12 = g[:, 2 * nh:3 * nh]
        g22 = g[:, 3 * nh:4 * nh]
        p11 = jnp.where(g11 > 0, g11, a1 * g11)
        p21 = jnp.where(g21 > 0, g21, a1 * g21)
        p12 = jnp.where(g12 > 0, g12, a2 * g12)
        p22 = jnp.where(g22 > 0, g22, a2 * g22)
        h1v = p11 + _H2 * p22
        h2v = p21 + _H1 * p12
        row = (t - nf) * bm
        h1_sc[pl.ds(row, bm), :] = h1v.astype(jnp.bfloat16)
        h2_sc[pl.ds(row, bm), :] = h2v.astype(jnp.bfloat16)
        part = jax.lax.dot_general(
            mskb_ref[0], h1v, (((1,), (0,)), ((), ())),
            preferred_element_type=jnp.float32)

        @pl.when(t == nf)
        def _():
            craw_sc[...] = jnp.zeros_like(craw_sc)

        craw_sc[...] += part

    @pl.when(t == nf + nm)
    def _():
        c = jax.nn.sigmoid(craw_sc[...] / jnp.sum(msk_ref[...]))
        v = jax.lax.dot_general(
            c, wd_ref[...], (((1,), (1,)), ((), ())),
            preferred_element_type=jnp.float32).astype(jnp.bfloat16)
        o1_ref[...] = jax.lax.dot_general(
            v, h1_sc[...], (((1,), (1,)), ((), ())),
            preferred_element_type=jnp.float32)
        o2_ref[...] = jax.lax.dot_general(
            v, h2_sc[...], (((1,), (1,)), ((), ())),
            preferred_element_type=jnp.float32)


def kernel(seq1, seq2, adj, sparse, training, msk, samp_bias1, samp_bias2,
           W1, b1, a1, W2, b2, a2, Wd, bd):
    n = seq1.shape[1]
    d = seq1.shape[2]
    nh = W1.shape[1]
    s1 = seq1[0]
    s2 = seq2[0]
    A = adj[0]

    bmf = n
    nf = 1
    bm = _blk(n, 400)
    nm = n // bm
    av1 = jnp.full((1, nh), a1, jnp.float32)
    av2 = jnp.full((1, nh), a2, jnp.float32)

    def _fidx(t):
        return (0, 0)

    def _aidx(t):
        return (jnp.minimum(jnp.maximum(t - nf, 0), nm - 1), 0)

    def _midx(t):
        return (jnp.minimum(jnp.maximum(t - nf, 0), nm - 1), 0, 0)

    o1, o2 = pl.pallas_call(
        functools.partial(_mega_kernel, nf, bmf, nm, bm),
        grid=(nf + nm + 1,),
        in_specs=[
            pl.BlockSpec((bmf, d), _fidx),
            pl.BlockSpec((bmf, d), _fidx),
            pl.BlockSpec((d, nh), lambda t: (0, 0)),
            pl.BlockSpec((d, nh), lambda t: (0, 0)),
            pl.BlockSpec((1, nh), lambda t: (0, 0)),
            pl.BlockSpec((1, nh), lambda t: (0, 0)),
            pl.BlockSpec((bm, n), _aidx),
            pl.BlockSpec((1, nh), lambda t: (0, 0)),
            pl.BlockSpec((1, nh), lambda t: (0, 0)),
            pl.BlockSpec((1, 1, bm), _midx),
            pl.BlockSpec((1, n), lambda t: (0, 0)),
            pl.BlockSpec((nh, nh), lambda t: (0, 0)),
        ],
        out_specs=[
            pl.BlockSpec((1, n), lambda t: (0, 0)),
            pl.BlockSpec((1, n), lambda t: (0, 0)),
        ],
        out_shape=[
            jax.ShapeDtypeStruct((1, n), jnp.float32),
            jax.ShapeDtypeStruct((1, n), jnp.float32),
        ],
        scratch_shapes=[
            pltpu.VMEM((n, 4 * nh), jnp.bfloat16),
            pltpu.VMEM((n, nh), jnp.bfloat16),
            pltpu.VMEM((n, nh), jnp.bfloat16),
            pltpu.VMEM((1, nh), jnp.float32),
        ],
        compiler_params=pltpu.CompilerParams(
            dimension_semantics=("arbitrary",)),
    )(s1, s2, W1, W2, b1.reshape(1, nh), b2.reshape(1, nh),
      A, av1, av2, msk.reshape(nm, 1, bm), msk, Wd)

    return jnp.concatenate(
        [o1 + samp_bias1 + bd[0], o2 + samp_bias2 + bd[0]], axis=1)


# speedup vs baseline: 1.3571x; 1.0165x over previous
import functools

import jax
import jax.numpy as jnp
from jax.experimental import pallas as pl
from jax.experimental.pallas import tpu as pltpu

_H1 = 0.5
_H2 = 0.5


def _blk(n, target):
    for b in range(min(target, n), 7, -1):
        if n % b == 0 and b % 8 == 0:
            return b
    return n


def _mega_kernel(nf, bmf, nm, bm,
                 s1_ref, s2_ref, w1_ref, w2_ref, b1_ref, b2_ref,
                 adj_ref, av1_ref, av2_ref, msk_ref, wd_ref,
                 o1_ref, o2_ref,
                 f_sc, h1_sc, h2_sc):
    t = pl.program_id(0)
    nh = av1_ref.shape[1]

    @pl.when(t < nf)
    def _():
        s1 = s1_ref[...]
        s2 = s2_ref[...]
        w1 = w1_ref[...]
        w2 = w2_ref[...]
        nhh = w1_ref.shape[1]
        row = t * bmf
        f_sc[pl.ds(row, bmf), 0 * nhh:1 * nhh] = (
            jnp.dot(s1, w1, preferred_element_type=jnp.float32)
            + b1_ref[...]).astype(jnp.bfloat16)
        f_sc[pl.ds(row, bmf), 1 * nhh:2 * nhh] = (
            jnp.dot(s2, w1, preferred_element_type=jnp.float32)
            + b1_ref[...]).astype(jnp.bfloat16)
        f_sc[pl.ds(row, bmf), 2 * nhh:3 * nhh] = (
            jnp.dot(s1, w2, preferred_element_type=jnp.float32)
            + b2_ref[...]).astype(jnp.bfloat16)
        f_sc[pl.ds(row, bmf), 3 * nhh:4 * nhh] = (
            jnp.dot(s2, w2, preferred_element_type=jnp.float32)
            + b2_ref[...]).astype(jnp.bfloat16)

    @pl.when((t >= nf) & (t < nf + nm))
    def _():
        a = adj_ref[...].astype(jnp.bfloat16)
        g = jax.lax.dot_general(
            a, f_sc[...], (((1,), (0,)), ((), ())),
            preferred_element_type=jnp.float32)
        a1 = av1_ref[...]
        a2 = av2_ref[...]
        g11 = g[:, 0 * nh:1 * nh]
        g21 = g[:, 1 * nh:2 * nh]
        g12 = g[:, 2 * nh:3 * nh]
        g22 = g[:, 3 * nh:4 * nh]
        p11 = jnp.where(g11 > 0, g11, a1 * g11)
        p21 = jnp.where(g21 > 0, g21, a1 * g21)
        p12 = jnp.where(g12 > 0, g12, a2 * g12)
        p22 = jnp.where(g22 > 0, g22, a2 * g22)
        h1v = p11 + _H2 * p22
        h2v = p21 + _H1 * p12
        row = (t - nf) * bm
        h1_sc[pl.ds(row, bm), :] = h1v.astype(jnp.bfloat16)
        h2_sc[pl.ds(row, bm), :] = h2v.astype(jnp.bfloat16)

    @pl.when(t == nf + nm)
    def _():
        m16 = msk_ref[...].astype(jnp.bfloat16)
        craw = jax.lax.dot_general(
            m16, h1_sc[...], (((1,), (0,)), ((), ())),
            preferred_element_type=jnp.float32)
        c = jax.nn.sigmoid(craw / jnp.sum(msk_ref[...]))
        v = jax.lax.dot_general(
            c, wd_ref[...], (((1,), (1,)), ((), ())),
            preferred_element_type=jnp.float32).astype(jnp.bfloat16)
        o1_ref[...] = jax.lax.dot_general(
            v, h1_sc[...], (((1,), (1,)), ((), ())),
            preferred_element_type=jnp.float32)
        o2_ref[...] = jax.lax.dot_general(
            v, h2_sc[...], (((1,), (1,)), ((), ())),
            preferred_element_type=jnp.float32)


def kernel(seq1, seq2, adj, sparse, training, msk, samp_bias1, samp_bias2,
           W1, b1, a1, W2, b2, a2, Wd, bd):
    n = seq1.shape[1]
    d = seq1.shape[2]
    nh = W1.shape[1]
    s1 = seq1[0]
    s2 = seq2[0]
    A = adj[0]

    bmf = n
    nf = 1
    bm = _blk(n, 400)
    nm = n // bm
    av1 = jnp.full((1, nh), a1, jnp.float32)
    av2 = jnp.full((1, nh), a2, jnp.float32)

    def _fidx(t):
        return (0, 0)

    def _aidx(t):
        return (jnp.minimum(jnp.maximum(t - nf, 0), nm - 1), 0)

    o1, o2 = pl.pallas_call(
        functools.partial(_mega_kernel, nf, bmf, nm, bm),
        grid=(nf + nm + 1,),
        in_specs=[
            pl.BlockSpec((bmf, d), _fidx),
            pl.BlockSpec((bmf, d), _fidx),
            pl.BlockSpec((d, nh), lambda t: (0, 0)),
            pl.BlockSpec((d, nh), lambda t: (0, 0)),
            pl.BlockSpec((1, nh), lambda t: (0, 0)),
            pl.BlockSpec((1, nh), lambda t: (0, 0)),
            pl.BlockSpec((bm, n), _aidx),
            pl.BlockSpec((1, nh), lambda t: (0, 0)),
            pl.BlockSpec((1, nh), lambda t: (0, 0)),
            pl.BlockSpec((1, n), lambda t: (0, 0)),
            pl.BlockSpec((nh, nh), lambda t: (0, 0)),
        ],
        out_specs=[
            pl.BlockSpec((1, n), lambda t: (0, 0)),
            pl.BlockSpec((1, n), lambda t: (0, 0)),
        ],
        out_shape=[
            jax.ShapeDtypeStruct((1, n), jnp.float32),
            jax.ShapeDtypeStruct((1, n), jnp.float32),
        ],
        scratch_shapes=[
            pltpu.VMEM((n, 4 * nh), jnp.bfloat16),
            pltpu.VMEM((n, nh), jnp.bfloat16),
            pltpu.VMEM((n, nh), jnp.bfloat16),
        ],
        compiler_params=pltpu.CompilerParams(
            dimension_semantics=("arbitrary",)),
    )(s1, s2, W1, W2, b1.reshape(1, nh), b2.reshape(1, nh),
      A, av1, av2, msk, Wd)

    return jnp.concatenate(
        [o1 + samp_bias1 + bd[0], o2 + samp_bias2 + bd[0]], axis=1)
